# baseline (speedup 1.0000x reference)
import functools

import jax
import jax.numpy as jnp
import numpy as np
from jax.experimental import pallas as pl
from jax.experimental.pallas import tpu as pltpu

ROWS = 64
COLS = 1_000_000
BC = 8192
NB = (COLS + BC - 1) // BC
ROWG = 32

_K0 = np.uint32(0)
_K1 = np.uint32(42)
_K2 = np.uint32(0 ^ 42 ^ 0x1BD11BDA)
_ROT1 = (13, 15, 26, 6)
_ROT2 = (17, 29, 16, 24)


def _rounds(x0, x1, rots):
    for r in rots:
        x0 = x0 + x1
        x1 = (x1 << np.uint32(r)) | (x1 >> np.uint32(32 - r))
        x1 = x0 ^ x1
    return x0, x1


def _threefry_bits(n):
    x0 = jnp.zeros_like(n) + _K0
    x1 = n + _K1
    x0, x1 = _rounds(x0, x1, _ROT1)
    x0, x1 = x0 + _K1, x1 + _K2 + np.uint32(1)
    x0, x1 = _rounds(x0, x1, _ROT2)
    x0, x1 = x0 + _K2, x1 + _K0 + np.uint32(2)
    x0, x1 = _rounds(x0, x1, _ROT1)
    x0, x1 = x0 + _K0, x1 + _K1 + np.uint32(3)
    x0, x1 = _rounds(x0, x1, _ROT2)
    x0, x1 = x0 + _K1, x1 + _K2 + np.uint32(4)
    x0, x1 = _rounds(x0, x1, _ROT1)
    x0, x1 = x0 + _K2, x1 + _K0 + np.uint32(5)
    return x0 ^ x1


def _sampler_kernel(logits_ref, out_ref, bv_ref, bc_ref):
    g = pl.program_id(0)
    j = pl.program_id(1)

    col = j * BC + jax.lax.broadcasted_iota(jnp.int32, (ROWG, BC), 1)
    row = g * ROWG + jax.lax.broadcasted_iota(jnp.int32, (ROWG, BC), 0)
    n = (row * COLS + col).astype(jnp.uint32)

    bits = _threefry_bits(n)
    fb = (bits >> np.uint32(9)) | np.uint32(0x3F800000)
    u = jax.lax.bitcast_convert_type(fb, jnp.float32) - jnp.float32(1.0)
    u = u * (jnp.float32(1.0) - jnp.float32(1e-20)) + jnp.float32(1e-20)
    u = jnp.maximum(jnp.float32(1e-20), u)

    gumbel = -jnp.log(-jnp.log(u))
    val = logits_ref[...] + gumbel
    val = jnp.where(col < COLS, val, -jnp.inf)

    @pl.when(j == 0)
    def _init():
        bv_ref[...] = val
        bc_ref[...] = col

    @pl.when(j != 0)
    def _update():
        upd = val > bv_ref[...]
        bv_ref[...] = jnp.where(upd, val, bv_ref[...])
        bc_ref[...] = jnp.where(upd, col, bc_ref[...])

    @pl.when(j == NB - 1)
    def _finalize():
        bv = bv_ref[...]
        bc = bc_ref[...]
        rowmax = jnp.max(bv, axis=1, keepdims=True)
        cand = jnp.where(bv == rowmax, bc, jnp.int32(2**30))
        out_ref[0] = jnp.min(cand, axis=1, keepdims=True)


@jax.jit
def kernel(logits):
    out = pl.pallas_call(
        _sampler_kernel,
        grid=(ROWS // ROWG, NB),
        in_specs=[pl.BlockSpec((ROWG, BC), lambda g, j: (g, j))],
        out_specs=pl.BlockSpec((1, ROWG, 1), lambda g, j: (g, 0, 0)),
        out_shape=jax.ShapeDtypeStruct((ROWS // ROWG, ROWG, 1), jnp.int32),
        scratch_shapes=[
            pltpu.VMEM((ROWG, BC), jnp.float32),
            pltpu.VMEM((ROWG, BC), jnp.int32),
        ],
        compiler_params=pltpu.CompilerParams(
            dimension_semantics=("parallel", "arbitrary"),
        ),
    )(logits)
    return out.reshape(ROWS)


# speedup vs baseline: 1.7602x; 1.7602x over previous
import jax
import jax.numpy as jnp
import numpy as np
from jax.experimental import pallas as pl
from jax.experimental.pallas import tpu as pltpu

ROWS = 64
COLS = 1_000_000
BC = 8192
NB = (COLS + BC - 1) // BC
ROWG = 32
CW = 256
NC = BC // CW

_K1 = np.uint32(42)
_K2 = np.uint32(0 ^ 42 ^ 0x1BD11BDA)
_ROT1 = (13, 15, 26, 6)
_ROT2 = (17, 29, 16, 24)


def _rounds(x0, x1, rots):
    for r in rots:
        x0 = x0 + x1
        x1 = (x1 << np.uint32(r)) | (x1 >> np.uint32(32 - r))
        x1 = x0 ^ x1
    return x0, x1


def _threefry_bits(x1):
    x0 = x1
    x1 = ((x1 << np.uint32(13)) | (x1 >> np.uint32(19))) ^ x0
    x0, x1 = _rounds(x0, x1, _ROT1[1:])
    x0, x1 = x0 + _K1, x1 + _K2 + np.uint32(1)
    x0, x1 = _rounds(x0, x1, _ROT2)
    x0, x1 = x0 + _K2, x1 + np.uint32(2)
    x0, x1 = _rounds(x0, x1, _ROT1)
    x0, x1 = x0, x1 + _K1 + np.uint32(3)
    x0, x1 = _rounds(x0, x1, _ROT2)
    x0, x1 = x0 + _K1, x1 + _K2 + np.uint32(4)
    x0, x1 = _rounds(x0, x1, _ROT1)
    x0, x1 = x0 + _K2, x1 + np.uint32(5)
    return x0 ^ x1


def _sampler_kernel(logits_ref, out_ref, bv_ref, bc_ref):
    g = pl.program_id(0)
    j = pl.program_id(1)

    iota_r = jax.lax.broadcasted_iota(jnp.int32, (ROWG, CW), 0)
    iota_c = jax.lax.broadcasted_iota(jnp.int32, (ROWG, CW), 1)
    rowbase42 = ((g * ROWG + iota_r) * COLS + 42).astype(jnp.uint32)
    colbase = j * BC + iota_c

    @pl.when(j == 0)
    def _init():
        bv_ref[...] = jnp.full((ROWG, CW), -jnp.inf, jnp.float32)
        bc_ref[...] = jnp.zeros((ROWG, CW), jnp.int32)

    bv = bv_ref[...]
    bc = bc_ref[...]
    for k in range(NC):
        col = colbase + (k * CW)
        bits = _threefry_bits(rowbase42 + col.astype(jnp.uint32))
        fb = (bits >> np.uint32(9)) | np.uint32(0x3F800000)
        u = jax.lax.bitcast_convert_type(fb, jnp.float32) - jnp.float32(1.0)
        u = u * (jnp.float32(1.0) - jnp.float32(1e-20)) + jnp.float32(1e-20)
        u = jnp.maximum(jnp.float32(1e-20), u)
        gumbel = -jnp.log(-jnp.log(u))
        val = logits_ref[:, k * CW:(k + 1) * CW] + gumbel
        val = jnp.where(col < COLS, val, -jnp.inf)
        upd = val > bv
        bv = jnp.where(upd, val, bv)
        bc = jnp.where(upd, col, bc)
    bv_ref[...] = bv
    bc_ref[...] = bc

    @pl.when(j == NB - 1)
    def _finalize():
        rowmax = jnp.max(bv, axis=1, keepdims=True)
        cand = jnp.where(bv == rowmax, bc, jnp.int32(2**30))
        out_ref[0] = jnp.min(cand, axis=1, keepdims=True)


@jax.jit
def kernel(logits):
    out = pl.pallas_call(
        _sampler_kernel,
        grid=(ROWS // ROWG, NB),
        in_specs=[pl.BlockSpec((ROWG, BC), lambda g, j: (g, j))],
        out_specs=pl.BlockSpec((1, ROWG, 1), lambda g, j: (g, 0, 0)),
        out_shape=jax.ShapeDtypeStruct((ROWS // ROWG, ROWG, 1), jnp.int32),
        scratch_shapes=[
            pltpu.VMEM((ROWG, CW), jnp.float32),
            pltpu.VMEM((ROWG, CW), jnp.int32),
        ],
        compiler_params=pltpu.CompilerParams(
            dimension_semantics=("parallel", "arbitrary"),
        ),
    )(logits)
    return out.reshape(ROWS)


# speedup vs baseline: 1.7627x; 1.0014x over previous
import math

import jax
import jax.numpy as jnp
import numpy as np
from jax.experimental import pallas as pl
from jax.experimental.pallas import tpu as pltpu
from jax.sharding import Mesh, PartitionSpec as P

try:
    from jax import shard_map as _shard_map_fn

    def _shard_map(f, mesh, in_specs, out_specs):
        return _shard_map_fn(f, mesh=mesh, in_specs=in_specs,
                             out_specs=out_specs, check_vma=False)
except ImportError:
    from jax.experimental.shard_map import shard_map as _shard_map_old

    def _shard_map(f, mesh, in_specs, out_specs):
        return _shard_map_old(f, mesh=mesh, in_specs=in_specs, out_specs=out_specs)

ROWS = 64
COLS = 1_000_000
BC = 8192
NB = (COLS + BC - 1) // BC
CW = 256
NC = BC // CW

_ALL_DEVS = jax.devices()
_NDEV = len(_ALL_DEVS) if ROWS % max(len(_ALL_DEVS), 1) == 0 else 1
_LROWS = ROWS // _NDEV
_ROWG = min(32, _LROWS)

_K1 = np.uint32(42)
_K2 = np.uint32(0 ^ 42 ^ 0x1BD11BDA)
_ROT1 = (13, 15, 26, 6)
_ROT2 = (17, 29, 16, 24)


def _rounds(x0, x1, rots):
    for r in rots:
        x0 = x0 + x1
        x1 = (x1 << np.uint32(r)) | (x1 >> np.uint32(32 - r))
        x1 = x0 ^ x1
    return x0, x1


def _threefry_bits(x1):
    x0 = x1
    x1 = ((x1 << np.uint32(13)) | (x1 >> np.uint32(19))) ^ x0
    x0, x1 = _rounds(x0, x1, _ROT1[1:])
    x0, x1 = x0 + _K1, x1 + _K2 + np.uint32(1)
    x0, x1 = _rounds(x0, x1, _ROT2)
    x0, x1 = x0 + _K2, x1 + np.uint32(2)
    x0, x1 = _rounds(x0, x1, _ROT1)
    x0, x1 = x0, x1 + _K1 + np.uint32(3)
    x0, x1 = _rounds(x0, x1, _ROT2)
    x0, x1 = x0 + _K1, x1 + _K2 + np.uint32(4)
    x0, x1 = _rounds(x0, x1, _ROT1)
    x0, x1 = x0 + _K2, x1 + np.uint32(5)
    return x0 ^ x1


_LN2 = np.float32(np.log(2.0))


def _sampler_kernel(rowbase_ref, logits_ref, out_ref, bv_ref, bc_ref):
    j = pl.program_id(1)

    iota_c = jax.lax.broadcasted_iota(jnp.int32, (_ROWG, CW), 1)
    rowbase42 = jnp.broadcast_to(rowbase_ref[...], (_ROWG, CW)).astype(jnp.uint32)
    colbase = j * BC + iota_c

    @pl.when(j == 0)
    def _init():
        bv_ref[...] = jnp.full((_ROWG, CW), -jnp.inf, jnp.float32)
        bc_ref[...] = jnp.zeros((_ROWG, CW), jnp.int32)

    bv = bv_ref[...]
    bc = bc_ref[...]
    for k in range(NC):
        col = colbase + (k * CW)
        bits = _threefry_bits(rowbase42 + col.astype(jnp.uint32))
        fb = (bits >> np.uint32(9)) | np.uint32(0x3F800000)
        u = jax.lax.bitcast_convert_type(fb, jnp.float32) - jnp.float32(1.0)
        u = jnp.maximum(u, jnp.float32(1e-20))
        gumbel = jnp.log2(jnp.log2(u) * (-_LN2)) * (-_LN2)
        val = logits_ref[:, k * CW:(k + 1) * CW] + gumbel
        val = jnp.where(col < COLS, val, -jnp.inf)
        upd = val > bv
        bv = jnp.where(upd, val, bv)
        bc = jnp.where(upd, col, bc)
    bv_ref[...] = bv
    bc_ref[...] = bc

    @pl.when(j == NB - 1)
    def _finalize():
        rowmax = jnp.max(bv, axis=1, keepdims=True)
        cand = jnp.where(bv == rowmax, bc, jnp.int32(2**30))
        out_ref[...] = jnp.min(cand, axis=1, keepdims=True)


def _run_shard(rowbase, logits_shard):
    out = pl.pallas_call(
        _sampler_kernel,
        grid=(_LROWS // _ROWG, NB),
        in_specs=[
            pl.BlockSpec((_ROWG, 1), lambda g, j: (g, 0)),
            pl.BlockSpec((_ROWG, BC), lambda g, j: (g, j)),
        ],
        out_specs=pl.BlockSpec((_ROWG, 1), lambda g, j: (g, 0)),
        out_shape=jax.ShapeDtypeStruct((_LROWS, 1), jnp.int32),
        scratch_shapes=[
            pltpu.VMEM((_ROWG, CW), jnp.float32),
            pltpu.VMEM((_ROWG, CW), jnp.int32),
        ],
        compiler_params=pltpu.CompilerParams(
            dimension_semantics=("arbitrary", "arbitrary"),
        ),
    )(rowbase, logits_shard)
    return out.reshape(_LROWS)


@jax.jit
def kernel(logits):
    rowbase = (jnp.arange(ROWS, dtype=jnp.int32) * COLS + 42).reshape(ROWS, 1)
    if _NDEV == 1:
        return _run_shard(rowbase, logits)
    mesh = Mesh(np.array(_ALL_DEVS[:_NDEV]), ("x",))
    f = _shard_map(
        _run_shard,
        mesh,
        (P("x", None), P("x", None)),
        P("x"),
    )
    return f(rowbase, logits)


# speedup vs baseline: 3.1919x; 1.8108x over previous
import math

import jax
import jax.numpy as jnp
import numpy as np
from jax.experimental import pallas as pl
from jax.experimental.pallas import tpu as pltpu
from jax.sharding import Mesh, PartitionSpec as P

try:
    from jax import shard_map as _shard_map_fn

    def _shard_map(f, mesh, in_specs, out_specs):
        return _shard_map_fn(f, mesh=mesh, in_specs=in_specs,
                             out_specs=out_specs, check_vma=False)
except ImportError:
    from jax.experimental.shard_map import shard_map as _shard_map_old

    def _shard_map(f, mesh, in_specs, out_specs):
        return _shard_map_old(f, mesh=mesh, in_specs=in_specs, out_specs=out_specs)

ROWS = 64
COLS = 1_000_000
BC = 8192
NB = (COLS + BC - 1) // BC
CW = 256
NC = BC // CW

_ALL_DEVS = jax.devices()
_NDEV = len(_ALL_DEVS) if ROWS % max(len(_ALL_DEVS), 1) == 0 else 1
_LROWS = ROWS // _NDEV
_ROWG = min(32, _LROWS)

_K1 = np.uint32(42)
_K2 = np.uint32(0 ^ 42 ^ 0x1BD11BDA)
_ROT1 = (13, 15, 26, 6)
_ROT2 = (17, 29, 16, 24)


def _rounds(x0, x1, rots):
    for r in rots:
        x0 = x0 + x1
        x1 = (x1 << np.uint32(r)) | (x1 >> np.uint32(32 - r))
        x1 = x0 ^ x1
    return x0, x1


def _threefry_bits(x1):
    x0 = x1
    x1 = ((x1 << np.uint32(13)) | (x1 >> np.uint32(19))) ^ x0
    x0, x1 = _rounds(x0, x1, _ROT1[1:])
    x0, x1 = x0 + _K1, x1 + _K2 + np.uint32(1)
    x0, x1 = _rounds(x0, x1, _ROT2)
    x0, x1 = x0 + _K2, x1 + np.uint32(2)
    x0, x1 = _rounds(x0, x1, _ROT1)
    x0, x1 = x0, x1 + _K1 + np.uint32(3)
    x0, x1 = _rounds(x0, x1, _ROT2)
    x0, x1 = x0 + _K1, x1 + _K2 + np.uint32(4)
    x0, x1 = _rounds(x0, x1, _ROT1)
    x0, x1 = x0 + _K2, x1 + np.uint32(5)
    return x0 ^ x1


_LN2 = np.float32(np.log(2.0))


def _sampler_kernel(off_ref, rowbase_ref, logits_ref, out_ref, bv_ref, bc_ref):
    del off_ref
    j = pl.program_id(1)

    iota_c = jax.lax.broadcasted_iota(jnp.int32, (_ROWG, CW), 1)
    rowbase42 = jnp.broadcast_to(rowbase_ref[...], (_ROWG, CW)).astype(jnp.uint32)
    colbase = j * BC + iota_c

    @pl.when(j == 0)
    def _init():
        bv_ref[...] = jnp.full((_ROWG, CW), -jnp.inf, jnp.float32)
        bc_ref[...] = jnp.zeros((_ROWG, CW), jnp.int32)

    bv = bv_ref[...]
    bc = bc_ref[...]
    for k in range(NC):
        col = colbase + (k * CW)
        bits = _threefry_bits(rowbase42 + col.astype(jnp.uint32))
        fb = (bits >> np.uint32(9)) | np.uint32(0x3F800000)
        u = jax.lax.bitcast_convert_type(fb, jnp.float32) - jnp.float32(1.0)
        u = jnp.maximum(u, jnp.float32(1e-20))
        gumbel = jnp.log2(jnp.log2(u) * (-_LN2)) * (-_LN2)
        val = logits_ref[:, k * CW:(k + 1) * CW] + gumbel
        val = jnp.where(col < COLS, val, -jnp.inf)
        upd = val > bv
        bv = jnp.where(upd, val, bv)
        bc = jnp.where(upd, col, bc)
    bv_ref[...] = bv
    bc_ref[...] = bc

    @pl.when(j == NB - 1)
    def _finalize():
        rowmax = jnp.max(bv, axis=1, keepdims=True)
        cand = jnp.where(bv == rowmax, bc, jnp.int32(2**30))
        out_ref[...] = jnp.min(cand, axis=1, keepdims=True)


def _run_shard(off, rowbase, logits):
    out = pl.pallas_call(
        _sampler_kernel,
        grid_spec=pltpu.PrefetchScalarGridSpec(
            num_scalar_prefetch=1,
            grid=(_LROWS // _ROWG, NB),
            in_specs=[
                pl.BlockSpec((_ROWG, 1), lambda g, j, off: (off[0] + g, 0)),
                pl.BlockSpec((_ROWG, BC), lambda g, j, off: (off[0] + g, j)),
            ],
            out_specs=pl.BlockSpec((_ROWG, 1), lambda g, j, off: (g, 0)),
            scratch_shapes=[
                pltpu.VMEM((_ROWG, CW), jnp.float32),
                pltpu.VMEM((_ROWG, CW), jnp.int32),
            ],
        ),
        out_shape=jax.ShapeDtypeStruct((_LROWS, 1), jnp.int32),
        compiler_params=pltpu.CompilerParams(
            dimension_semantics=("arbitrary", "arbitrary"),
        ),
    )(off, rowbase, logits)
    return out.reshape(_LROWS)


@jax.jit
def kernel(logits):
    rowbase = (jnp.arange(ROWS, dtype=jnp.int32) * COLS + 42).reshape(ROWS, 1)
    if _NDEV == 1:
        return _run_shard(jnp.zeros((1,), jnp.int32), rowbase, logits)

    def _body(rb, lg):
        ai = jax.lax.axis_index("x")
        off = (ai * (_LROWS // _ROWG)).astype(jnp.int32).reshape(1)
        return _run_shard(off, rb, lg)

    mesh = Mesh(np.array(_ALL_DEVS[:_NDEV]), ("x",))
    f = _shard_map(
        _body,
        mesh,
        (P(None, None), P(None, None)),
        P("x"),
    )
    return f(rowbase, logits)
